# baseline (speedup 1.0000x reference)
import functools

import jax
import jax.numpy as jnp
from jax import lax
from jax.experimental import pallas as pl
from jax.experimental.pallas import tpu as pltpu
from jax.experimental.pallas import tpu_sc as plsc

N_NODES = 10000
N_PAD = 10240
N_EDGES = 320000
E_PAD = 327680
INPUT_DIM = 128
HIDDEN = 256
HALF = 128
N_EXTRA = 64

NUM_SUBCORES = 16
ROWS_PER_TILE = N_PAD // NUM_SUBCORES
EDGES_PER_TILE = E_PAD // NUM_SUBCORES
KROWS = 4
NSUPER = EDGES_PER_TILE // (KROWS * 128)
IDXROWS_PER_TILE = EDGES_PER_TILE // 128
DEG_EDGES_PER_TILE = E_PAD // 32
DEG_IDXROWS = DEG_EDGES_PER_TILE // 128

TC_BLK = 1024
TC_GRID = N_PAD // TC_BLK


def _prep_body(ws_ref, wg_ref, bs_ref, bg_ref, wenc_ref, benc_ref,
               w1_ref, b0_ref):
    wenc_s = wenc_ref[0:HIDDEN, :]
    wenc_g = wenc_ref[HIDDEN:2 * HIDDEN, :]
    w1_ref[...] = (jnp.dot(ws_ref[...], wenc_s, preferred_element_type=jnp.float32)
                   + jnp.dot(wg_ref[...], wenc_g, preferred_element_type=jnp.float32))
    b0_ref[...] = (jnp.dot(bs_ref[...], wenc_s, preferred_element_type=jnp.float32)
                   + jnp.dot(bg_ref[...], wenc_g, preferred_element_type=jnp.float32)
                   + benc_ref[...])


def _prep(ws, wg, bs2, bg2, wenc, benc2):
    return pl.pallas_call(
        _prep_body,
        out_shape=(jax.ShapeDtypeStruct((INPUT_DIM, HIDDEN), jnp.float32),
                   jax.ShapeDtypeStruct((1, HIDDEN), jnp.float32)),
    )(ws, wg, bs2, bg2, wenc, benc2)


def _stage1_body(x_ref, pos_ref, w1_ref, wenc_ref, b0_ref, h2_ref):
    w2 = wenc_ref[2 * HIDDEN:2 * HIDDEN + N_EXTRA, :]
    h = (jnp.dot(x_ref[...], w1_ref[...], preferred_element_type=jnp.float32)
         + jnp.dot(pos_ref[...], w2, preferred_element_type=jnp.float32)
         + b0_ref[...])
    h2_ref[0] = h[:, :HALF]
    h2_ref[1] = h[:, HALF:]


def _stage1(xp, posp, w1, wenc, b0):
    return pl.pallas_call(
        _stage1_body,
        grid=(TC_GRID,),
        in_specs=[
            pl.BlockSpec((TC_BLK, INPUT_DIM), lambda i: (i, 0)),
            pl.BlockSpec((TC_BLK, N_EXTRA), lambda i: (i, 0)),
            pl.BlockSpec((INPUT_DIM, HIDDEN), lambda i: (0, 0)),
            pl.BlockSpec((2 * HIDDEN + N_EXTRA, HIDDEN), lambda i: (0, 0)),
            pl.BlockSpec((1, HIDDEN), lambda i: (0, 0)),
        ],
        out_specs=pl.BlockSpec((2, TC_BLK, HALF), lambda i: (0, i, 0)),
        out_shape=jax.ShapeDtypeStruct((2, N_PAD, HALF), jnp.float32),
        compiler_params=pltpu.CompilerParams(
            dimension_semantics=("parallel",)),
    )(xp, posp, w1, wenc, b0)


EROW = 16
NBUF = 10
IDXR = 32
NSUPER = EDGES_PER_TILE // (IDXR * EROW)
ROWS32_PER_TILE = EDGES_PER_TILE // EROW


def _sc_body(h2, src2, dst2,
             g2,
             src_v, dst_v, rows_v, gsh,
             sem0, sem1, sem2, sem3, sem4, sem5, sem6, sem7, sem8, sem9):
    c = lax.axis_index("c")
    s = lax.axis_index("s")
    r0 = s * ROWS_PER_TILE
    sems = [sem0, sem1, sem2, sem3, sem4, sem5, sem6, sem7, sem8, sem9]

    def zrow(i, carry):
        for k in range(HALF // 16):
            rows_v[0, i, pl.ds(k * 16, 16)] = jnp.zeros((16,), jnp.float32)
            rows_v[1, i, pl.ds(k * 16, 16)] = jnp.zeros((16,), jnp.float32)
        return carry

    lax.fori_loop(0, EROW, zrow, 0)
    for t in range(ROWS_PER_TILE // (2 * EROW)):
        pltpu.sync_copy(rows_v.at[0],
                        gsh.at[pl.ds(r0 + 2 * t * EROW, EROW)])
        pltpu.sync_copy(rows_v.at[1],
                        gsh.at[pl.ds(r0 + (2 * t + 1) * EROW, EROW)])

    plsc.subcore_barrier()

    idx0 = s * ROWS32_PER_TILE

    def superblock(i, carry):
        r = idx0 + i * IDXR
        pltpu.sync_copy(src2.at[pl.ds(r, IDXR)], src_v)
        pltpu.sync_copy(dst2.at[pl.ds(r, IDXR)], dst_v)
        for j in range(NBUF):
            pltpu.async_copy(h2.at[c].at[src_v.at[j]], rows_v.at[j], sems[j])
        for j in range(IDXR):
            b = j % NBUF
            pltpu.make_async_copy(h2.at[c].at[src_v.at[j]],
                                  rows_v.at[b], sems[b]).wait()
            pltpu.sync_copy(rows_v.at[b], gsh.at[dst_v.at[j]], add=True)
            if j + NBUF < IDXR:
                pltpu.async_copy(h2.at[c].at[src_v.at[j + NBUF]],
                                 rows_v.at[b], sems[b])
        return carry

    lax.fori_loop(0, NSUPER, superblock, 0)

    plsc.subcore_barrier()

    pltpu.sync_copy(gsh.at[pl.ds(r0, ROWS_PER_TILE)],
                    g2.at[c].at[pl.ds(r0, ROWS_PER_TILE)])


@functools.lru_cache(maxsize=1)
def _make_sc_call():
    mesh = plsc.VectorSubcoreMesh(core_axis_name="c", subcore_axis_name="s",
                                  num_cores=2, num_subcores=NUM_SUBCORES)
    return pl.kernel(
        _sc_body,
        out_type=jax.ShapeDtypeStruct((2, N_PAD, HALF), jnp.float32),
        mesh=mesh,
        scratch_types=[
            pltpu.VMEM((IDXR, EROW), jnp.int32),
            pltpu.VMEM((IDXR, EROW), jnp.int32),
            pltpu.VMEM((NBUF, EROW, HALF), jnp.float32),
            pltpu.VMEM_SHARED((N_PAD, HALF), jnp.float32),
            pltpu.SemaphoreType.DMA,
            pltpu.SemaphoreType.DMA,
            pltpu.SemaphoreType.DMA,
            pltpu.SemaphoreType.DMA,
            pltpu.SemaphoreType.DMA,
            pltpu.SemaphoreType.DMA,
            pltpu.SemaphoreType.DMA,
            pltpu.SemaphoreType.DMA,
            pltpu.SemaphoreType.DMA,
            pltpu.SemaphoreType.DMA,
        ],
    )


DEG_IDXROWS = E_PAD // EROW // 32


def _deg_body(dst2, degout, dst_v, deg_v):
    c = lax.axis_index("c")
    s = lax.axis_index("s")
    wid = c * NUM_SUBCORES + s

    def zrow(i, carry):
        deg_v[pl.ds(i * 16, 16)] = jnp.zeros((16,), jnp.float32)
        return carry

    lax.fori_loop(0, N_PAD // 16, zrow, 0)

    pltpu.sync_copy(dst2.at[pl.ds(wid * DEG_IDXROWS, DEG_IDXROWS)], dst_v)

    ones = jnp.ones((16,), jnp.float32)

    def accum(i, carry):
        for k in range(EROW // 16):
            idx = dst_v[i, pl.ds(k * 16, 16)]
            plsc.addupdate_scatter(deg_v, [idx], ones)
        return carry

    lax.fori_loop(0, DEG_IDXROWS, accum, 0)

    pltpu.sync_copy(deg_v, degout.at[wid])


@functools.lru_cache(maxsize=1)
def _make_deg_call():
    mesh = plsc.VectorSubcoreMesh(core_axis_name="c", subcore_axis_name="s",
                                  num_cores=2, num_subcores=NUM_SUBCORES)
    return pl.kernel(
        _deg_body,
        out_type=jax.ShapeDtypeStruct((32, N_PAD), jnp.float32),
        mesh=mesh,
        compiler_params=pltpu.CompilerParams(needs_layout_passes=False),
        scratch_types=[
            pltpu.VMEM((DEG_IDXROWS, EROW), jnp.int32),
            pltpu.VMEM((N_PAD,), jnp.float32),
        ],
    )


def _stage2_body(g2_ref, h2_ref, deg_ref, wmsg_ref, bmsg_ref, out_ref):
    gcat = jnp.concatenate([g2_ref[0], g2_ref[1]], axis=-1)
    hcat = jnp.concatenate([h2_ref[0], h2_ref[1]], axis=-1)
    deg = jnp.sum(deg_ref[...], axis=0)[:, None]
    agg = (jnp.dot(gcat, wmsg_ref[...], preferred_element_type=jnp.float32)
           + deg * bmsg_ref[...])
    out_ref[...] = jnp.maximum(agg + hcat, 0.0)


def _stage2(g2, h2, degp, wmsg, bmsg2):
    return pl.pallas_call(
        _stage2_body,
        grid=(TC_GRID,),
        in_specs=[
            pl.BlockSpec((2, TC_BLK, HALF), lambda i: (0, i, 0)),
            pl.BlockSpec((2, TC_BLK, HALF), lambda i: (0, i, 0)),
            pl.BlockSpec((32, TC_BLK), lambda i: (0, i)),
            pl.BlockSpec((HIDDEN, HIDDEN), lambda i: (0, 0)),
            pl.BlockSpec((1, HIDDEN), lambda i: (0, 0)),
        ],
        out_specs=pl.BlockSpec((TC_BLK, HIDDEN), lambda i: (i, 0)),
        out_shape=jax.ShapeDtypeStruct((N_PAD, HIDDEN), jnp.float32),
        compiler_params=pltpu.CompilerParams(
            dimension_semantics=("parallel",)),
    )(g2, h2, degp, wmsg, bmsg2)


def kernel(x, edge_index, pos, Wg, bg, Ws, bs, Wenc, benc, Wmsg, bmsg):
    n = x.shape[0]
    ei = edge_index.astype(jnp.int32)
    src = jnp.concatenate(
        [ei[0], jnp.zeros((E_PAD - N_EDGES,), jnp.int32)]
    ).reshape(E_PAD // EROW, EROW)
    dst = jnp.concatenate(
        [ei[1], jnp.full((E_PAD - N_EDGES,), N_PAD - 1, jnp.int32)]
    ).reshape(E_PAD // EROW, EROW)
    xp = jnp.pad(x, ((0, N_PAD - n), (0, 0)))
    posp = jnp.pad(pos, ((0, N_PAD - n), (0, 0)))

    w1, b0 = _prep(Ws, Wg, bs.reshape(1, -1), bg.reshape(1, -1),
                   Wenc, benc.reshape(1, -1))
    h2 = _stage1(xp, posp, w1, Wenc, b0)

    degp = _make_deg_call()(dst)
    g2 = _make_sc_call()(h2, src, dst)

    out = _stage2(g2, h2, degp, Wmsg, bmsg.reshape(1, -1))
    return out[:n]


# speedup vs baseline: 1.0812x; 1.0812x over previous
import functools

import jax
import jax.numpy as jnp
from jax import lax
from jax.experimental import pallas as pl
from jax.experimental.pallas import tpu as pltpu
from jax.experimental.pallas import tpu_sc as plsc

N_NODES = 10000
N_PAD = 10240
N_EDGES = 320000
E_PAD = 327680
INPUT_DIM = 128
HIDDEN = 256
HALF = 128
N_EXTRA = 64

NUM_SUBCORES = 16
ROWS_PER_TILE = N_PAD // NUM_SUBCORES
EDGES_PER_TILE = E_PAD // NUM_SUBCORES
KROWS = 4
NSUPER = EDGES_PER_TILE // (KROWS * 128)
IDXROWS_PER_TILE = EDGES_PER_TILE // 128
DEG_EDGES_PER_TILE = E_PAD // 32
DEG_IDXROWS = DEG_EDGES_PER_TILE // 128

TC_BLK = 1024
TC_GRID = N_PAD // TC_BLK


def _prep_body(ws_ref, wg_ref, bs_ref, bg_ref, wenc_ref, benc_ref,
               w1_ref, b0_ref):
    wenc_s = wenc_ref[0:HIDDEN, :]
    wenc_g = wenc_ref[HIDDEN:2 * HIDDEN, :]
    w1_ref[...] = (jnp.dot(ws_ref[...], wenc_s, preferred_element_type=jnp.float32)
                   + jnp.dot(wg_ref[...], wenc_g, preferred_element_type=jnp.float32))
    b0_ref[...] = (jnp.dot(bs_ref[...], wenc_s, preferred_element_type=jnp.float32)
                   + jnp.dot(bg_ref[...], wenc_g, preferred_element_type=jnp.float32)
                   + benc_ref[...])


def _prep(ws, wg, bs2, bg2, wenc, benc2):
    return pl.pallas_call(
        _prep_body,
        out_shape=(jax.ShapeDtypeStruct((INPUT_DIM, HIDDEN), jnp.float32),
                   jax.ShapeDtypeStruct((1, HIDDEN), jnp.float32)),
    )(ws, wg, bs2, bg2, wenc, benc2)


def _stage1_body(x_ref, pos_ref, w1_ref, wenc_ref, b0_ref, h2_ref):
    w2 = wenc_ref[2 * HIDDEN:2 * HIDDEN + N_EXTRA, :]
    h = (jnp.dot(x_ref[...], w1_ref[...], preferred_element_type=jnp.float32)
         + jnp.dot(pos_ref[...], w2, preferred_element_type=jnp.float32)
         + b0_ref[...])
    h2_ref[0] = h[:, :HALF]
    h2_ref[1] = h[:, HALF:]


def _stage1(xp, posp, w1, wenc, b0):
    return pl.pallas_call(
        _stage1_body,
        grid=(TC_GRID,),
        in_specs=[
            pl.BlockSpec((TC_BLK, INPUT_DIM), lambda i: (i, 0)),
            pl.BlockSpec((TC_BLK, N_EXTRA), lambda i: (i, 0)),
            pl.BlockSpec((INPUT_DIM, HIDDEN), lambda i: (0, 0)),
            pl.BlockSpec((2 * HIDDEN + N_EXTRA, HIDDEN), lambda i: (0, 0)),
            pl.BlockSpec((1, HIDDEN), lambda i: (0, 0)),
        ],
        out_specs=pl.BlockSpec((2, TC_BLK, HALF), lambda i: (0, i, 0)),
        out_shape=jax.ShapeDtypeStruct((2, N_PAD, HALF), jnp.float32),
        compiler_params=pltpu.CompilerParams(
            dimension_semantics=("parallel",)),
    )(xp, posp, w1, wenc, b0)


EROW = 32
NBUF = 5
IDXR = 16
NSUPER = EDGES_PER_TILE // (IDXR * EROW)
ROWS32_PER_TILE = EDGES_PER_TILE // EROW


def _sc_body(h2, src2, dst2,
             g2,
             src_v, dst_v, rows_v, gsh,
             sem0, sem1, sem2, sem3, sem4, semA, semB):
    c = lax.axis_index("c")
    s = lax.axis_index("s")
    r0 = s * ROWS_PER_TILE
    sems = [sem0, sem1, sem2, sem3, sem4]
    isems = [semA, semB]

    def zrow(i, carry):
        for k in range(HALF // 16):
            rows_v[0, i, pl.ds(k * 16, 16)] = jnp.zeros((16,), jnp.float32)
            rows_v[1, i, pl.ds(k * 16, 16)] = jnp.zeros((16,), jnp.float32)
        return carry

    lax.fori_loop(0, EROW, zrow, 0)
    nz = ROWS_PER_TILE // EROW
    for t in range(nz):
        pltpu.async_copy(rows_v.at[t % 2],
                         gsh.at[pl.ds(r0 + t * EROW, EROW)], isems[t % 2])
    for t in range(nz):
        pltpu.make_async_copy(rows_v.at[t % 2],
                              gsh.at[pl.ds(r0 + t * EROW, EROW)],
                              isems[t % 2]).wait()

    plsc.subcore_barrier()

    idx0 = s * ROWS32_PER_TILE

    def proc(slot, sb):
        for j in range(NBUF):
            pltpu.async_copy(h2.at[c].at[src_v.at[slot, j]],
                             rows_v.at[j], sems[j])
        for j in range(IDXR):
            b = j % NBUF
            pltpu.make_async_copy(h2.at[c].at[src_v.at[slot, j]],
                                  rows_v.at[b], sems[b]).wait()
            pltpu.sync_copy(rows_v.at[b], gsh.at[dst_v.at[slot, j]], add=True)
            if j + NBUF < IDXR:
                pltpu.async_copy(h2.at[c].at[src_v.at[slot, j + NBUF]],
                                 rows_v.at[b], sems[b])

    def load_idx(slot, sb, sem):
        r = idx0 + sb * IDXR
        pltpu.async_copy(src2.at[pl.ds(r, IDXR)], src_v.at[slot], sem)
        pltpu.async_copy(dst2.at[pl.ds(r, IDXR)], dst_v.at[slot], sem)

    def wait_idx(slot, sb, sem):
        r = idx0 + sb * IDXR
        pltpu.make_async_copy(src2.at[pl.ds(r, IDXR)], src_v.at[slot], sem).wait()
        pltpu.make_async_copy(dst2.at[pl.ds(r, IDXR)], dst_v.at[slot], sem).wait()

    load_idx(0, 0, semA)

    def pair(i, carry):
        sb = 2 * i
        wait_idx(0, sb, semA)
        load_idx(1, sb + 1, semB)
        proc(0, sb)
        wait_idx(1, sb + 1, semB)

        @pl.when(i + 1 < NSUPER // 2)
        def _():
            load_idx(0, sb + 2, semA)

        proc(1, sb + 1)
        return carry

    lax.fori_loop(0, NSUPER // 2, pair, 0)

    plsc.subcore_barrier()

    pltpu.sync_copy(gsh.at[pl.ds(r0, ROWS_PER_TILE)],
                    g2.at[c].at[pl.ds(r0, ROWS_PER_TILE)])


@functools.lru_cache(maxsize=1)
def _make_sc_call():
    mesh = plsc.VectorSubcoreMesh(core_axis_name="c", subcore_axis_name="s",
                                  num_cores=2, num_subcores=NUM_SUBCORES)
    return pl.kernel(
        _sc_body,
        out_type=jax.ShapeDtypeStruct((2, N_PAD, HALF), jnp.float32),
        mesh=mesh,
        scratch_types=[
            pltpu.VMEM((2, IDXR, EROW), jnp.int32),
            pltpu.VMEM((2, IDXR, EROW), jnp.int32),
            pltpu.VMEM((NBUF, EROW, HALF), jnp.float32),
            pltpu.VMEM_SHARED((N_PAD, HALF), jnp.float32),
            pltpu.SemaphoreType.DMA,
            pltpu.SemaphoreType.DMA,
            pltpu.SemaphoreType.DMA,
            pltpu.SemaphoreType.DMA,
            pltpu.SemaphoreType.DMA,
            pltpu.SemaphoreType.DMA,
            pltpu.SemaphoreType.DMA,
        ],
    )


DEG_IDXROWS = E_PAD // EROW // 32


def _deg_body(dst2, degout, dst_v, deg_v):
    c = lax.axis_index("c")
    s = lax.axis_index("s")
    wid = c * NUM_SUBCORES + s

    def zrow(i, carry):
        deg_v[pl.ds(i * 16, 16)] = jnp.zeros((16,), jnp.float32)
        return carry

    lax.fori_loop(0, N_PAD // 16, zrow, 0)

    pltpu.sync_copy(dst2.at[pl.ds(wid * DEG_IDXROWS, DEG_IDXROWS)], dst_v)

    ones = jnp.ones((16,), jnp.float32)

    def accum(i, carry):
        for k in range(EROW // 16):
            idx = dst_v[i, pl.ds(k * 16, 16)]
            plsc.addupdate_scatter(deg_v, [idx], ones)
        return carry

    lax.fori_loop(0, DEG_IDXROWS, accum, 0)

    pltpu.sync_copy(deg_v, degout.at[wid])


@functools.lru_cache(maxsize=1)
def _make_deg_call():
    mesh = plsc.VectorSubcoreMesh(core_axis_name="c", subcore_axis_name="s",
                                  num_cores=2, num_subcores=NUM_SUBCORES)
    return pl.kernel(
        _deg_body,
        out_type=jax.ShapeDtypeStruct((32, N_PAD), jnp.float32),
        mesh=mesh,
        compiler_params=pltpu.CompilerParams(needs_layout_passes=False),
        scratch_types=[
            pltpu.VMEM((DEG_IDXROWS, EROW), jnp.int32),
            pltpu.VMEM((N_PAD,), jnp.float32),
        ],
    )


def _stage2_body(g2_ref, h2_ref, deg_ref, wmsg_ref, bmsg_ref, out_ref):
    gcat = jnp.concatenate([g2_ref[0], g2_ref[1]], axis=-1)
    hcat = jnp.concatenate([h2_ref[0], h2_ref[1]], axis=-1)
    deg = jnp.sum(deg_ref[...], axis=0)[:, None]
    agg = (jnp.dot(gcat, wmsg_ref[...], preferred_element_type=jnp.float32)
           + deg * bmsg_ref[...])
    out_ref[...] = jnp.maximum(agg + hcat, 0.0)


def _stage2(g2, h2, degp, wmsg, bmsg2):
    return pl.pallas_call(
        _stage2_body,
        grid=(TC_GRID,),
        in_specs=[
            pl.BlockSpec((2, TC_BLK, HALF), lambda i: (0, i, 0)),
            pl.BlockSpec((2, TC_BLK, HALF), lambda i: (0, i, 0)),
            pl.BlockSpec((32, TC_BLK), lambda i: (0, i)),
            pl.BlockSpec((HIDDEN, HIDDEN), lambda i: (0, 0)),
            pl.BlockSpec((1, HIDDEN), lambda i: (0, 0)),
        ],
        out_specs=pl.BlockSpec((TC_BLK, HIDDEN), lambda i: (i, 0)),
        out_shape=jax.ShapeDtypeStruct((N_PAD, HIDDEN), jnp.float32),
        compiler_params=pltpu.CompilerParams(
            dimension_semantics=("parallel",)),
    )(g2, h2, degp, wmsg, bmsg2)


def kernel(x, edge_index, pos, Wg, bg, Ws, bs, Wenc, benc, Wmsg, bmsg):
    n = x.shape[0]
    ei = edge_index.astype(jnp.int32)
    src = jnp.concatenate(
        [ei[0], jnp.zeros((E_PAD - N_EDGES,), jnp.int32)]
    ).reshape(E_PAD // EROW, EROW)
    dst = jnp.concatenate(
        [ei[1], jnp.full((E_PAD - N_EDGES,), N_PAD - 1, jnp.int32)]
    ).reshape(E_PAD // EROW, EROW)
    xp = jnp.pad(x, ((0, N_PAD - n), (0, 0)))
    posp = jnp.pad(pos, ((0, N_PAD - n), (0, 0)))

    w1, b0 = _prep(Ws, Wg, bs.reshape(1, -1), bg.reshape(1, -1),
                   Wenc, benc.reshape(1, -1))
    h2 = _stage1(xp, posp, w1, Wenc, b0)

    degp = _make_deg_call()(dst)
    g2 = _make_sc_call()(h2, src, dst)

    out = _stage2(g2, h2, degp, Wmsg, bmsg.reshape(1, -1))
    return out[:n]


# speedup vs baseline: 1.1679x; 1.0802x over previous
import functools

import jax
import jax.numpy as jnp
from jax import lax
from jax.experimental import pallas as pl
from jax.experimental.pallas import tpu as pltpu
from jax.experimental.pallas import tpu_sc as plsc

N_NODES = 10000
N_PAD = 10240
N_EDGES = 320000
E_PAD = 327680
INPUT_DIM = 128
HIDDEN = 256
HALF = 128
N_EXTRA = 64

NUM_SUBCORES = 16
ROWS_PER_TILE = N_PAD // NUM_SUBCORES
EDGES_PER_TILE = E_PAD // NUM_SUBCORES
KROWS = 4
NSUPER = EDGES_PER_TILE // (KROWS * 128)
IDXROWS_PER_TILE = EDGES_PER_TILE // 128
DEG_EDGES_PER_TILE = E_PAD // 32
DEG_IDXROWS = DEG_EDGES_PER_TILE // 128

TC_BLK = 1024
TC_GRID = N_PAD // TC_BLK


ROW_BLK = 1000


def _stage1_body(x_ref, pos_ref, ws_ref, wg_ref, bs_ref, bg_ref,
                 wenc_ref, benc_ref, h2_ref):
    wenc_s = wenc_ref[0:HIDDEN, :]
    wenc_g = wenc_ref[HIDDEN:2 * HIDDEN, :]
    w2 = wenc_ref[2 * HIDDEN:2 * HIDDEN + N_EXTRA, :]
    w1 = (jnp.dot(ws_ref[...], wenc_s, preferred_element_type=jnp.float32)
          + jnp.dot(wg_ref[...], wenc_g, preferred_element_type=jnp.float32))
    b0 = (jnp.dot(bs_ref[...], wenc_s, preferred_element_type=jnp.float32)
          + jnp.dot(bg_ref[...], wenc_g, preferred_element_type=jnp.float32)
          + benc_ref[...])
    h = (jnp.dot(x_ref[...], w1, preferred_element_type=jnp.float32)
         + jnp.dot(pos_ref[...], w2, preferred_element_type=jnp.float32)
         + b0)
    h2_ref[0] = h[:, :HALF]
    h2_ref[1] = h[:, HALF:]


def _stage1(x, pos, ws, wg, bs2, bg2, wenc, benc2):
    return pl.pallas_call(
        _stage1_body,
        grid=(N_NODES // ROW_BLK,),
        in_specs=[
            pl.BlockSpec((ROW_BLK, INPUT_DIM), lambda i: (i, 0)),
            pl.BlockSpec((ROW_BLK, N_EXTRA), lambda i: (i, 0)),
            pl.BlockSpec((INPUT_DIM, HIDDEN), lambda i: (0, 0)),
            pl.BlockSpec((INPUT_DIM, HIDDEN), lambda i: (0, 0)),
            pl.BlockSpec((1, HIDDEN), lambda i: (0, 0)),
            pl.BlockSpec((1, HIDDEN), lambda i: (0, 0)),
            pl.BlockSpec((2 * HIDDEN + N_EXTRA, HIDDEN), lambda i: (0, 0)),
            pl.BlockSpec((1, HIDDEN), lambda i: (0, 0)),
        ],
        out_specs=pl.BlockSpec((2, ROW_BLK, HALF), lambda i: (0, i, 0)),
        out_shape=jax.ShapeDtypeStruct((2, N_PAD, HALF), jnp.float32),
        compiler_params=pltpu.CompilerParams(
            dimension_semantics=("arbitrary",)),
    )(x, pos, ws, wg, bs2, bg2, wenc, benc2)


EROW = 32
NBUF = 5
IDXR = 16
NSUPER = EDGES_PER_TILE // (IDXR * EROW)
ROWS32_PER_TILE = EDGES_PER_TILE // EROW


def _sc_body(h2, src2, dst2,
             g2,
             src_v, dst_v, rows_v, gsh,
             sem0, sem1, sem2, sem3, sem4, semA, semB):
    c = lax.axis_index("c")
    s = lax.axis_index("s")
    r0 = s * ROWS_PER_TILE
    sems = [sem0, sem1, sem2, sem3, sem4]
    isems = [semA, semB]

    def zrow(i, carry):
        for k in range(HALF // 16):
            rows_v[0, i, pl.ds(k * 16, 16)] = jnp.zeros((16,), jnp.float32)
            rows_v[1, i, pl.ds(k * 16, 16)] = jnp.zeros((16,), jnp.float32)
        return carry

    lax.fori_loop(0, EROW, zrow, 0)
    nz = ROWS_PER_TILE // EROW
    for t in range(nz):
        pltpu.async_copy(rows_v.at[t % 2],
                         gsh.at[pl.ds(r0 + t * EROW, EROW)], isems[t % 2])
    for t in range(nz):
        pltpu.make_async_copy(rows_v.at[t % 2],
                              gsh.at[pl.ds(r0 + t * EROW, EROW)],
                              isems[t % 2]).wait()

    plsc.subcore_barrier()

    idx0 = s * ROWS32_PER_TILE

    def proc(slot, sb):
        for j in range(NBUF):
            pltpu.async_copy(h2.at[c].at[src_v.at[slot, j]],
                             rows_v.at[j], sems[j])
        for j in range(IDXR):
            b = j % NBUF
            pltpu.make_async_copy(h2.at[c].at[src_v.at[slot, j]],
                                  rows_v.at[b], sems[b]).wait()
            pltpu.sync_copy(rows_v.at[b], gsh.at[dst_v.at[slot, j]], add=True)
            if j + NBUF < IDXR:
                pltpu.async_copy(h2.at[c].at[src_v.at[slot, j + NBUF]],
                                 rows_v.at[b], sems[b])

    def load_idx(slot, sb, sem):
        r = idx0 + sb * IDXR
        pltpu.async_copy(src2.at[pl.ds(r, IDXR)], src_v.at[slot], sem)
        pltpu.async_copy(dst2.at[pl.ds(r, IDXR)], dst_v.at[slot], sem)

    def wait_idx(slot, sb, sem):
        r = idx0 + sb * IDXR
        pltpu.make_async_copy(src2.at[pl.ds(r, IDXR)], src_v.at[slot], sem).wait()
        pltpu.make_async_copy(dst2.at[pl.ds(r, IDXR)], dst_v.at[slot], sem).wait()

    load_idx(0, 0, semA)

    def pair(i, carry):
        sb = 2 * i
        wait_idx(0, sb, semA)
        load_idx(1, sb + 1, semB)
        proc(0, sb)
        wait_idx(1, sb + 1, semB)

        @pl.when(i + 1 < NSUPER // 2)
        def _():
            load_idx(0, sb + 2, semA)

        proc(1, sb + 1)
        return carry

    lax.fori_loop(0, NSUPER // 2, pair, 0)

    plsc.subcore_barrier()

    pltpu.sync_copy(gsh.at[pl.ds(r0, ROWS_PER_TILE)],
                    g2.at[c].at[pl.ds(r0, ROWS_PER_TILE)])


@functools.lru_cache(maxsize=1)
def _make_sc_call():
    mesh = plsc.VectorSubcoreMesh(core_axis_name="c", subcore_axis_name="s",
                                  num_cores=2, num_subcores=NUM_SUBCORES)
    return pl.kernel(
        _sc_body,
        out_type=jax.ShapeDtypeStruct((2, N_PAD, HALF), jnp.float32),
        mesh=mesh,
        scratch_types=[
            pltpu.VMEM((2, IDXR, EROW), jnp.int32),
            pltpu.VMEM((2, IDXR, EROW), jnp.int32),
            pltpu.VMEM((NBUF, EROW, HALF), jnp.float32),
            pltpu.VMEM_SHARED((N_PAD, HALF), jnp.float32),
            pltpu.SemaphoreType.DMA,
            pltpu.SemaphoreType.DMA,
            pltpu.SemaphoreType.DMA,
            pltpu.SemaphoreType.DMA,
            pltpu.SemaphoreType.DMA,
            pltpu.SemaphoreType.DMA,
            pltpu.SemaphoreType.DMA,
        ],
    )


DEG_IDXROWS = E_PAD // EROW // 32


def _deg_body(dst2, degout, dst_v, deg_v):
    c = lax.axis_index("c")
    s = lax.axis_index("s")
    wid = c * NUM_SUBCORES + s

    def zrow(i, carry):
        deg_v[pl.ds(i * 16, 16)] = jnp.zeros((16,), jnp.float32)
        return carry

    lax.fori_loop(0, N_PAD // 16, zrow, 0)

    pltpu.sync_copy(dst2.at[pl.ds(wid * DEG_IDXROWS, DEG_IDXROWS)], dst_v)

    ones = jnp.ones((16,), jnp.float32)

    def accum(i, carry):
        for k in range(EROW // 16):
            idx = dst_v[i, pl.ds(k * 16, 16)]
            plsc.addupdate_scatter(deg_v, [idx], ones)
        return carry

    lax.fori_loop(0, DEG_IDXROWS, accum, 0)

    pltpu.sync_copy(deg_v, degout.at[wid])


@functools.lru_cache(maxsize=1)
def _make_deg_call():
    mesh = plsc.VectorSubcoreMesh(core_axis_name="c", subcore_axis_name="s",
                                  num_cores=2, num_subcores=NUM_SUBCORES)
    return pl.kernel(
        _deg_body,
        out_type=jax.ShapeDtypeStruct((32, N_PAD), jnp.float32),
        mesh=mesh,
        compiler_params=pltpu.CompilerParams(needs_layout_passes=False),
        scratch_types=[
            pltpu.VMEM((DEG_IDXROWS, EROW), jnp.int32),
            pltpu.VMEM((N_PAD,), jnp.float32),
        ],
    )


def _stage2_body(g2_ref, h2_ref, deg_ref, wmsg_ref, bmsg_ref, out_ref):
    gcat = jnp.concatenate([g2_ref[0], g2_ref[1]], axis=-1)
    hcat = jnp.concatenate([h2_ref[0], h2_ref[1]], axis=-1)
    deg = jnp.sum(deg_ref[...], axis=0)[:, None]
    agg = (jnp.dot(gcat, wmsg_ref[...], preferred_element_type=jnp.float32)
           + deg * bmsg_ref[...])
    out_ref[...] = jnp.maximum(agg + hcat, 0.0)


def _stage2(g2, h2, degp, wmsg, bmsg2):
    return pl.pallas_call(
        _stage2_body,
        grid=(TC_GRID,),
        in_specs=[
            pl.BlockSpec((2, TC_BLK, HALF), lambda i: (0, i, 0)),
            pl.BlockSpec((2, TC_BLK, HALF), lambda i: (0, i, 0)),
            pl.BlockSpec((32, TC_BLK), lambda i: (0, i)),
            pl.BlockSpec((HIDDEN, HIDDEN), lambda i: (0, 0)),
            pl.BlockSpec((1, HIDDEN), lambda i: (0, 0)),
        ],
        out_specs=pl.BlockSpec((TC_BLK, HIDDEN), lambda i: (i, 0)),
        out_shape=jax.ShapeDtypeStruct((N_PAD, HIDDEN), jnp.float32),
        compiler_params=pltpu.CompilerParams(
            dimension_semantics=("parallel",)),
    )(g2, h2, degp, wmsg, bmsg2)


def kernel(x, edge_index, pos, Wg, bg, Ws, bs, Wenc, benc, Wmsg, bmsg):
    n = x.shape[0]
    ei = edge_index.astype(jnp.int32)
    src = jnp.concatenate(
        [ei[0], jnp.zeros((E_PAD - N_EDGES,), jnp.int32)]
    ).reshape(E_PAD // EROW, EROW)
    dst = jnp.concatenate(
        [ei[1], jnp.full((E_PAD - N_EDGES,), N_PAD - 1, jnp.int32)]
    ).reshape(E_PAD // EROW, EROW)
    h2 = _stage1(x, pos, Ws, Wg, bs.reshape(1, -1), bg.reshape(1, -1),
                 Wenc, benc.reshape(1, -1))

    degp = _make_deg_call()(dst)
    g2 = _make_sc_call()(h2, src, dst)

    out = _stage2(g2, h2, degp, Wmsg, bmsg.reshape(1, -1))
    return out[:n]
